# baseline (speedup 1.0000x reference)
import functools

import jax
import jax.numpy as jnp
from jax import lax
from jax.experimental import pallas as pl
from jax.experimental.pallas import tpu as pltpu
from jax.experimental.pallas import tpu_sc as plsc

B = 4096
F = 26
V = 100000
D = 32
TOT = F * D

NC = 2
NS = 16
NW = NC * NS
NG = 2
FG = F // NG


def _gather_body(tab_ref, sid_ref, out_ref, ids_v, col_v, sem_g):
    cpw = FG * D // NW
    wid = lax.axis_index("s") * NC + lax.axis_index("c")
    u0 = wid * cpw
    f0 = u0 // D
    f1 = (u0 + cpw - 1) // D
    pltpu.sync_copy(sid_ref.at[f0], ids_v.at[0])
    pltpu.sync_copy(sid_ref.at[f1], ids_v.at[1])

    def gather(k, b):
        u = u0 + k
        f = u // D
        c = u % D
        lane = jnp.where(f == f0, 0, 1)
        return pltpu.make_async_copy(
            tab_ref.at[f].at[c].at[ids_v.at[lane]], col_v.at[b], sem_g)

    gather(0, 0).start()
    for k in range(cpw):
        gather(k, k % 2).wait()
        if k + 1 < cpw:
            gather(k + 1, (k + 1) % 2).start()
        pltpu.sync_copy(col_v.at[k % 2], out_ref.at[u0 + k])


@functools.cache
def _make_gather():
    return pl.kernel(
        _gather_body,
        out_type=jax.ShapeDtypeStruct((FG * D, B), jnp.float32),
        mesh=plsc.VectorSubcoreMesh(core_axis_name="c", subcore_axis_name="s",
                                    num_cores=NC, num_subcores=NS),
        scratch_types=[
            pltpu.VMEM((2, B), jnp.int32),
            pltpu.VMEM((2, B), jnp.float32),
            pltpu.SemaphoreType.DMA,
        ],
        compiler_params=pltpu.CompilerParams(use_tc_tiling_on_sc=False),
    )


def _tower_body(xt_ref, g_ref, bb_ref, w1_ref, b1_ref, w2_ref, b2_ref,
                w3_ref, b3_ref, out_ref):
    xt = xt_ref[...]
    mu = jnp.mean(xt, axis=1, keepdims=True)
    xc = xt - mu
    var = jnp.mean(xc * xc, axis=1, keepdims=True)
    xn = xc * (g_ref[...] * lax.rsqrt(var + 1e-5)) + bb_ref[...]
    h = lax.dot_general(xn, w1_ref[...], (((0,), (0,)), ((), ())),
                        preferred_element_type=jnp.float32)
    h = jnp.maximum(h + b1_ref[...], 0.0)
    h = jnp.dot(h, w2_ref[...], preferred_element_type=jnp.float32)
    h = jnp.maximum(h + b2_ref[...], 0.0)
    out = jnp.dot(h, w3_ref[...], preferred_element_type=jnp.float32)
    out_ref[...] = out + b3_ref[...]


def _tower(xt, g, bb, w1, b1, w2, b2, w3, b3):
    return pl.pallas_call(
        _tower_body,
        out_shape=jax.ShapeDtypeStruct((B, 128), jnp.float32),
    )(xt, g, bb, w1, b1, w2, b2, w3, b3)


def kernel(sparse, tables, bn_gamma, bn_beta, W1, b1, W2, b2, W3, b3):
    tabt = jnp.transpose(tables, (0, 2, 1))
    sid = sparse.T
    gather = _make_gather()
    parts = []
    for g in range(NG):
        parts.append(gather(tabt[g * FG:(g + 1) * FG],
                            sid[g * FG:(g + 1) * FG]))
    xt = jnp.concatenate(parts, axis=0)
    return _tower(
        xt,
        bn_gamma.reshape(TOT, 1),
        bn_beta.reshape(TOT, 1),
        W1, b1.reshape(1, 512),
        W2, b2.reshape(1, 256),
        W3, b3.reshape(1, 128),
    )


# speedup vs baseline: 1.2442x; 1.2442x over previous
import functools

import jax
import jax.numpy as jnp
from jax import lax
from jax.experimental import pallas as pl
from jax.experimental.pallas import tpu as pltpu
from jax.experimental.pallas import tpu_sc as plsc

B = 4096
F = 26
V = 100000
D = 32
TOT = F * D

NC = 2
NS = 16
NW = NC * NS
NG = 1
FG = F // NG


def _gather_body(tab_ref, sid_ref, out_ref, ids_v, col_v, sem_g):
    cpw = FG * D // NW
    wid = lax.axis_index("s") * NC + lax.axis_index("c")
    u0 = wid * cpw
    f0 = u0 // D
    f1 = (u0 + cpw - 1) // D
    pltpu.sync_copy(sid_ref.at[f0], ids_v.at[0])
    pltpu.sync_copy(sid_ref.at[f1], ids_v.at[1])

    def gather(k, b):
        u = u0 + k
        f = u // D
        c = u % D
        lane = jnp.where(f == f0, 0, 1)
        return pltpu.make_async_copy(
            tab_ref.at[f].at[c].at[ids_v.at[lane]], col_v.at[b], sem_g)

    gather(0, 0).start()
    for k in range(cpw):
        gather(k, k % 2).wait()
        if k + 1 < cpw:
            gather(k + 1, (k + 1) % 2).start()
        pltpu.sync_copy(col_v.at[k % 2], out_ref.at[u0 + k])


@functools.cache
def _make_gather():
    return pl.kernel(
        _gather_body,
        out_type=jax.ShapeDtypeStruct((FG * D, B), jnp.float32),
        mesh=plsc.VectorSubcoreMesh(core_axis_name="c", subcore_axis_name="s",
                                    num_cores=NC, num_subcores=NS),
        scratch_types=[
            pltpu.VMEM((2, B), jnp.int32),
            pltpu.VMEM((2, B), jnp.float32),
            pltpu.SemaphoreType.DMA,
        ],
        compiler_params=pltpu.CompilerParams(use_tc_tiling_on_sc=False),
    )


def _tower_body(xt_ref, g_ref, bb_ref, w1_ref, b1_ref, w2_ref, b2_ref,
                w3_ref, b3_ref, out_ref):
    xt = xt_ref[...]
    mu = jnp.mean(xt, axis=1, keepdims=True)
    xc = xt - mu
    var = jnp.mean(xc * xc, axis=1, keepdims=True)
    xn = xc * (g_ref[...] * lax.rsqrt(var + 1e-5)) + bb_ref[...]
    h = lax.dot_general(xn, w1_ref[...], (((0,), (0,)), ((), ())),
                        preferred_element_type=jnp.float32)
    h = jnp.maximum(h + b1_ref[...], 0.0)
    h = jnp.dot(h, w2_ref[...], preferred_element_type=jnp.float32)
    h = jnp.maximum(h + b2_ref[...], 0.0)
    out = jnp.dot(h, w3_ref[...], preferred_element_type=jnp.float32)
    out_ref[...] = out + b3_ref[...]


def _tower(xt, g, bb, w1, b1, w2, b2, w3, b3):
    return pl.pallas_call(
        _tower_body,
        out_shape=jax.ShapeDtypeStruct((B, 128), jnp.float32),
    )(xt, g, bb, w1, b1, w2, b2, w3, b3)


def kernel(sparse, tables, bn_gamma, bn_beta, W1, b1, W2, b2, W3, b3):
    tabt = jnp.transpose(tables, (0, 2, 1))
    sid = sparse.T
    gather = _make_gather()
    parts = []
    for g in range(NG):
        parts.append(gather(tabt[g * FG:(g + 1) * FG],
                            sid[g * FG:(g + 1) * FG]))
    xt = jnp.concatenate(parts, axis=0)
    return _tower(
        xt,
        bn_gamma.reshape(TOT, 1),
        bn_beta.reshape(TOT, 1),
        W1, b1.reshape(1, 512),
        W2, b2.reshape(1, 256),
        W3, b3.reshape(1, 128),
    )
